# baseline (speedup 1.0000x reference)
import functools

import jax
import jax.numpy as jnp
from jax import lax
from jax.experimental import pallas as pl
from jax.experimental.pallas import tpu as pltpu
from jax.experimental.pallas import tpu_sc as plsc

N_NODES = 10000
D = 128
N_EDGES = 320000
EPS = 1e-5

_R = 2000
_G = N_NODES // _R

_NC = 2
_NS = 16
_CH = 64
_CPT = 160
_NCHUNK = _CPT * _NC * _NS
_EPAD = _NCHUNK * _CH
_NPAD = 10240
_RPT = _NPAD // _NS
_ZB = 64


def _fc1_body(x_ref, w_ref, b_ref, h0_ref, st_ref):
    i = pl.program_id(0)
    h0 = jnp.dot(x_ref[...], w_ref[...], preferred_element_type=jnp.float32)
    h0 = h0 + b_ref[...]
    h0_ref[...] = h0

    @pl.when(i == 0)
    def _():
        st_ref[...] = jnp.zeros_like(st_ref)

    st_ref[0:1, :] += jnp.sum(h0, axis=0, keepdims=True)
    st_ref[1:2, :] += jnp.sum(h0 * h0, axis=0, keepdims=True)


def _proj_body(h0_ref, st_ref, g_ref, be_ref, wr_ref, wn_ref, bgc_ref,
               hr_ref, m_ref):
    st = st_ref[...]
    mean = st[0:1, :] * (1.0 / N_NODES)
    var = st[1:2, :] * (1.0 / N_NODES) - mean * mean
    a = g_ref[...] * lax.rsqrt(var + EPS)
    c = be_ref[...] - mean * a
    h = h0_ref[...] * a + c
    hr_ref[...] = jnp.dot(h, wr_ref[...],
                          preferred_element_type=jnp.float32) + bgc_ref[...]
    m_ref[...] = jnp.dot(h, wn_ref[...], preferred_element_type=jnp.float32)


def _gc_body(hr_ref, a0_ref, a1_ref, w2_ref, b2_ref, t_ref, st_ref):
    i = pl.program_id(0)
    gc = hr_ref[...] + a0_ref[0] + a1_ref[0]
    g = gc * 0.5 * (1.0 + lax.erf(gc * 0.7071067811865476))
    t = jnp.dot(g, w2_ref[...], preferred_element_type=jnp.float32) + b2_ref[...]
    t_ref[...] = t

    @pl.when(i == 0)
    def _():
        st_ref[...] = jnp.zeros_like(st_ref)

    st_ref[0:1, :] += jnp.sum(t, axis=0, keepdims=True)
    st_ref[1:2, :] += jnp.sum(t * t, axis=0, keepdims=True)


def _fin_body(t_ref, st_ref, g_ref, be_ref, x_ref, o_ref):
    st = st_ref[...]
    mean = st[0:1, :] * (1.0 / N_NODES)
    var = st[1:2, :] * (1.0 / N_NODES) - mean * mean
    a = g_ref[...] * lax.rsqrt(var + EPS)
    c = be_ref[...] - mean * a
    o_ref[...] = t_ref[...] * a + c + x_ref[...]


_row_spec = pl.BlockSpec((_R, D), lambda i: (i, 0))
_full_spec = pl.BlockSpec((D, D), lambda i: (0, 0))
_vec_spec = pl.BlockSpec((1, D), lambda i: (0, 0))
_st_spec = pl.BlockSpec((8, D), lambda i: (0, 0))
_rows_out = jax.ShapeDtypeStruct((N_NODES, D), jnp.float32)
_st_out = jax.ShapeDtypeStruct((8, D), jnp.float32)


_fc1 = pl.pallas_call(
    _fc1_body, grid=(_G,),
    in_specs=[_row_spec, _full_spec, _vec_spec],
    out_specs=[_row_spec, _st_spec],
    out_shape=[_rows_out, _st_out],
)

_proj = pl.pallas_call(
    _proj_body, grid=(_G,),
    in_specs=[_row_spec, _st_spec, _vec_spec, _vec_spec, _full_spec,
              _full_spec, _vec_spec],
    out_specs=[_row_spec, _row_spec],
    out_shape=[_rows_out, _rows_out],
)

_gc = pl.pallas_call(
    _gc_body, grid=(_G,),
    in_specs=[_row_spec,
              pl.BlockSpec((1, _R, D), lambda i: (0, i, 0)),
              pl.BlockSpec((1, _R, D), lambda i: (1, i, 0)),
              _full_spec, _vec_spec],
    out_specs=[_row_spec, _st_spec],
    out_shape=[_rows_out, _st_out],
)

_fin = pl.pallas_call(
    _fin_body, grid=(_G,),
    in_specs=[_row_spec, _st_spec, _vec_spec, _vec_spec, _row_spec],
    out_specs=_row_spec,
    out_shape=_rows_out,
)


def _sc_body(m_hbm, sd_hbm, out_hbm,
             sdbuf, rows, zbuf,
             agg_sh, gsem, gsem1, isem):
    cid = lax.axis_index("c")
    sid = lax.axis_index("s")
    wid = cid * _NS + sid
    r0e = wid * _CPT
    row0 = sid * _RPT

    pltpu.make_async_copy(sd_hbm.at[pl.ds(r0e, _CPT)], sdbuf, isem).start()

    z16 = jnp.zeros((16,), jnp.float32)

    def _zrow(r, carry):
        for j in range(D // 16):
            zbuf[r, pl.ds(j * 16, 16)] = z16
        return carry

    lax.fori_loop(0, _ZB, _zrow, 0)

    pltpu.make_async_copy(sd_hbm.at[pl.ds(r0e, _CPT)], sdbuf, isem).wait()

    def _g(c, r_ref, sem):
        return pltpu.make_async_copy(
            m_hbm.at[sdbuf.at[c, pl.ds(0, _CH)]], r_ref, sem)

    def _s(c, r_ref):
        pltpu.sync_copy(r_ref, agg_sh.at[sdbuf.at[c, pl.ds(_CH, _CH)]],
                        add=True)

    _g(0, rows, gsem).start()

    for t in range(_RPT // _ZB):
        pltpu.make_async_copy(
            zbuf.at[pl.ds(0, _ZB)],
            agg_sh.at[pl.ds(row0 + t * _ZB, _ZB)], gsem1).start()
    for t in range(_RPT // _ZB):
        pltpu.make_async_copy(
            zbuf.at[pl.ds(0, _ZB)],
            agg_sh.at[pl.ds(row0 + t * _ZB, _ZB)], gsem1).wait()
    plsc.subcore_barrier()

    _g(1, zbuf, gsem1).start()

    def _pair(p, carry):
        c = 2 * p
        _g(c, rows, gsem).wait()
        _s(c, rows)

        @pl.when(c + 2 < _CPT)
        def _():
            _g(c + 2, rows, gsem).start()

        _g(c + 1, zbuf, gsem1).wait()
        _s(c + 1, zbuf)

        @pl.when(c + 3 < _CPT)
        def _():
            _g(c + 3, zbuf, gsem1).start()

        return carry

    lax.fori_loop(0, _CPT // 2, _pair, 0)

    plsc.subcore_barrier()

    for t in range(_RPT // _ZB):
        r0 = row0 + t * _ZB
        pltpu.make_async_copy(agg_sh.at[pl.ds(r0, _ZB)],
                              out_hbm.at[cid, pl.ds(r0, _ZB)], gsem).start()
    for t in range(_RPT // _ZB):
        r0 = row0 + t * _ZB
        pltpu.make_async_copy(agg_sh.at[pl.ds(r0, _ZB)],
                              out_hbm.at[cid, pl.ds(r0, _ZB)], gsem).wait()


@functools.cache
def _make_segsum():
    return functools.partial(
        pl.kernel,
        mesh=plsc.VectorSubcoreMesh(core_axis_name="c", subcore_axis_name="s"),
        out_type=jax.ShapeDtypeStruct((_NC, _NPAD, D), jnp.float32),
        scratch_types=[
            pltpu.VMEM((_CPT, 2 * _CH), jnp.int32),
            pltpu.VMEM((_CH, D), jnp.float32),
            pltpu.VMEM((_CH, D), jnp.float32),
            pltpu.VMEM_SHARED((_NPAD, D), jnp.float32),
            pltpu.SemaphoreType.DMA,
            pltpu.SemaphoreType.DMA,
            pltpu.SemaphoreType.DMA,
        ],
    )(_sc_body)


def kernel(x, edge_index, W1, b1, g1, be1, Wr, Wn, bgc, W2, b2, g2, be2):
    ei = edge_index.astype(jnp.int32)
    npad = _EPAD - N_EDGES

    def _tile_chunks(idx):
        a = idx.reshape(_CPT, _NC * _NS, _CH)
        return a.transpose(1, 0, 2).reshape(_NCHUNK, _CH)

    src = _tile_chunks(jnp.concatenate([ei[0], jnp.zeros((npad,), jnp.int32)]))
    pad_dst = N_NODES + jnp.arange(npad, dtype=jnp.int32) % (_NPAD - N_NODES)
    dst = _tile_chunks(jnp.concatenate([ei[1], pad_dst]))
    sd = jnp.concatenate([src, dst], axis=1)
    b1r = b1.reshape(1, D)
    g1r = g1.reshape(1, D)
    be1r = be1.reshape(1, D)
    bgcr = bgc.reshape(1, D)
    b2r = b2.reshape(1, D)
    g2r = g2.reshape(1, D)
    be2r = be2.reshape(1, D)

    h0, st1 = _fc1(x, W1, b1r)
    hr, m = _proj(h0, st1, g1r, be1r, Wr, Wn, bgcr)
    aggp = _make_segsum()(m, sd)
    t, st2 = _gc(hr, aggp, aggp, W2, b2r)
    return _fin(t, st2, g2r, be2r, x)


# speedup vs baseline: 1.0637x; 1.0637x over previous
import functools

import jax
import jax.numpy as jnp
from jax import lax
from jax.experimental import pallas as pl
from jax.experimental.pallas import tpu as pltpu
from jax.experimental.pallas import tpu_sc as plsc

N_NODES = 10000
D = 128
N_EDGES = 320000
EPS = 1e-5

_R = 2000
_G = N_NODES // _R

_NC = 2
_NS = 16
_CH = 128
_CPT = 80
_NP = _CPT // 2
_NCHUNK = _CPT * _NC * _NS
_EPAD = _NCHUNK * _CH
_NPAD = 10240
_RPT = _NPAD // _NS
_ZB = 128


def _fc1_body(x_ref, w_ref, b_ref, h0_ref, st_ref):
    i = pl.program_id(0)
    h0 = jnp.dot(x_ref[...], w_ref[...], preferred_element_type=jnp.float32)
    h0 = h0 + b_ref[...]
    h0_ref[...] = h0

    @pl.when(i == 0)
    def _():
        st_ref[...] = jnp.zeros_like(st_ref)

    st_ref[0:1, :] += jnp.sum(h0, axis=0, keepdims=True)
    st_ref[1:2, :] += jnp.sum(h0 * h0, axis=0, keepdims=True)


def _proj_body(h0_ref, st_ref, g_ref, be_ref, wr_ref, wn_ref, bgc_ref,
               hr_ref, m_ref):
    st = st_ref[...]
    mean = st[0:1, :] * (1.0 / N_NODES)
    var = st[1:2, :] * (1.0 / N_NODES) - mean * mean
    a = g_ref[...] * lax.rsqrt(var + EPS)
    c = be_ref[...] - mean * a
    h = h0_ref[...] * a + c
    hr_ref[...] = jnp.dot(h, wr_ref[...],
                          preferred_element_type=jnp.float32) + bgc_ref[...]
    m_ref[...] = jnp.dot(h, wn_ref[...], preferred_element_type=jnp.float32)


def _gc_body(hr_ref, a0_ref, a1_ref, w2_ref, b2_ref, t_ref, st_ref):
    i = pl.program_id(0)
    gc = hr_ref[...] + a0_ref[0] + a1_ref[0]
    g = gc * 0.5 * (1.0 + lax.erf(gc * 0.7071067811865476))
    t = jnp.dot(g, w2_ref[...], preferred_element_type=jnp.float32) + b2_ref[...]
    t_ref[...] = t

    @pl.when(i == 0)
    def _():
        st_ref[...] = jnp.zeros_like(st_ref)

    st_ref[0:1, :] += jnp.sum(t, axis=0, keepdims=True)
    st_ref[1:2, :] += jnp.sum(t * t, axis=0, keepdims=True)


def _fin_body(t_ref, st_ref, g_ref, be_ref, x_ref, o_ref):
    st = st_ref[...]
    mean = st[0:1, :] * (1.0 / N_NODES)
    var = st[1:2, :] * (1.0 / N_NODES) - mean * mean
    a = g_ref[...] * lax.rsqrt(var + EPS)
    c = be_ref[...] - mean * a
    o_ref[...] = t_ref[...] * a + c + x_ref[...]


_row_spec = pl.BlockSpec((_R, D), lambda i: (i, 0))
_full_spec = pl.BlockSpec((D, D), lambda i: (0, 0))
_vec_spec = pl.BlockSpec((1, D), lambda i: (0, 0))
_st_spec = pl.BlockSpec((8, D), lambda i: (0, 0))
_rows_out = jax.ShapeDtypeStruct((N_NODES, D), jnp.float32)
_st_out = jax.ShapeDtypeStruct((8, D), jnp.float32)


_fc1 = pl.pallas_call(
    _fc1_body, grid=(_G,),
    in_specs=[_row_spec, _full_spec, _vec_spec],
    out_specs=[_row_spec, _st_spec],
    out_shape=[_rows_out, _st_out],
)

_proj = pl.pallas_call(
    _proj_body, grid=(_G,),
    in_specs=[_row_spec, _st_spec, _vec_spec, _vec_spec, _full_spec,
              _full_spec, _vec_spec],
    out_specs=[_row_spec, _row_spec],
    out_shape=[_rows_out, _rows_out],
)

_gc = pl.pallas_call(
    _gc_body, grid=(_G,),
    in_specs=[_row_spec,
              pl.BlockSpec((1, _R, D), lambda i: (0, i, 0)),
              pl.BlockSpec((1, _R, D), lambda i: (1, i, 0)),
              _full_spec, _vec_spec],
    out_specs=[_row_spec, _st_spec],
    out_shape=[_rows_out, _st_out],
)

_fin = pl.pallas_call(
    _fin_body, grid=(_G,),
    in_specs=[_row_spec, _st_spec, _vec_spec, _vec_spec, _row_spec],
    out_specs=_row_spec,
    out_shape=_rows_out,
)


def _sc_body(m_hbm, sd_hbm, out_hbm,
             idxa, idxb, rows, zbuf,
             agg_sh, gsem, gsem1, isem):
    cid = lax.axis_index("c")
    sid = lax.axis_index("s")
    wid = cid * _NS + sid
    q0 = wid * _NP
    row0 = sid * _RPT

    def _ld(q, i_ref):
        return pltpu.make_async_copy(sd_hbm.at[q0 + q], i_ref, isem)

    def _g(i_ref, j, r_ref, sem):
        return pltpu.make_async_copy(
            m_hbm.at[i_ref.at[j, pl.ds(0, _CH)]], r_ref, sem)

    def _s(i_ref, j, r_ref):
        pltpu.sync_copy(r_ref, agg_sh.at[i_ref.at[j, pl.ds(_CH, _CH)]],
                        add=True)

    _ld(0, idxa).start()

    z16 = jnp.zeros((16,), jnp.float32)

    def _zrow(r, carry):
        for j in range(D // 16):
            zbuf[r, pl.ds(j * 16, 16)] = z16
        return carry

    lax.fori_loop(0, _ZB, _zrow, 0)

    _ld(0, idxa).wait()
    _g(idxa, 0, rows, gsem).start()
    _ld(1, idxb).start()

    for t in range(_RPT // _ZB):
        pltpu.make_async_copy(
            zbuf, agg_sh.at[pl.ds(row0 + t * _ZB, _ZB)], gsem1).start()
    for t in range(_RPT // _ZB):
        pltpu.make_async_copy(
            zbuf, agg_sh.at[pl.ds(row0 + t * _ZB, _ZB)], gsem1).wait()
    plsc.subcore_barrier()

    _g(idxa, 1, zbuf, gsem1).start()

    def _pbody(p, cur, nxt):
        _g(cur, 0, rows, gsem).wait()
        _s(cur, 0, rows)

        @pl.when(p + 1 < _NP)
        def _():
            _ld(p + 1, nxt).wait()
            _g(nxt, 0, rows, gsem).start()

        _g(cur, 1, zbuf, gsem1).wait()
        _s(cur, 1, zbuf)

        @pl.when(p + 1 < _NP)
        def _():
            _g(nxt, 1, zbuf, gsem1).start()

        @pl.when(p + 2 < _NP)
        def _():
            _ld(p + 2, cur).start()

    def _pair(p, carry):
        @pl.when(p % 2 == 0)
        def _():
            _pbody(p, idxa, idxb)

        @pl.when(p % 2 == 1)
        def _():
            _pbody(p, idxb, idxa)

        return carry

    lax.fori_loop(0, _NP, _pair, 0)

    plsc.subcore_barrier()

    for t in range(_RPT // _ZB):
        r0 = row0 + t * _ZB
        pltpu.make_async_copy(agg_sh.at[pl.ds(r0, _ZB)],
                              out_hbm.at[cid, pl.ds(r0, _ZB)], gsem).start()
    for t in range(_RPT // _ZB):
        r0 = row0 + t * _ZB
        pltpu.make_async_copy(agg_sh.at[pl.ds(r0, _ZB)],
                              out_hbm.at[cid, pl.ds(r0, _ZB)], gsem).wait()


@functools.cache
def _make_segsum():
    return functools.partial(
        pl.kernel,
        mesh=plsc.VectorSubcoreMesh(core_axis_name="c", subcore_axis_name="s"),
        out_type=jax.ShapeDtypeStruct((_NC, _NPAD, D), jnp.float32),
        scratch_types=[
            pltpu.VMEM((2, 2 * _CH), jnp.int32),
            pltpu.VMEM((2, 2 * _CH), jnp.int32),
            pltpu.VMEM((_CH, D), jnp.float32),
            pltpu.VMEM((_CH, D), jnp.float32),
            pltpu.VMEM_SHARED((_NPAD, D), jnp.float32),
            pltpu.SemaphoreType.DMA,
            pltpu.SemaphoreType.DMA,
            pltpu.SemaphoreType.DMA,
        ],
    )(_sc_body)


def kernel(x, edge_index, W1, b1, g1, be1, Wr, Wn, bgc, W2, b2, g2, be2):
    ei = edge_index.astype(jnp.int32)
    npad = _EPAD - N_EDGES

    def _tile_chunks(idx):
        a = idx.reshape(_CPT, _NC * _NS, _CH)
        return a.transpose(1, 0, 2).reshape(_NCHUNK, _CH)

    src = _tile_chunks(jnp.concatenate([ei[0], jnp.zeros((npad,), jnp.int32)]))
    pad_dst = N_NODES + jnp.arange(npad, dtype=jnp.int32) % (_NPAD - N_NODES)
    dst = _tile_chunks(jnp.concatenate([ei[1], pad_dst]))
    sd = jnp.concatenate([src, dst], axis=1).reshape(_NCHUNK // 2, 2, 2 * _CH)
    b1r = b1.reshape(1, D)
    g1r = g1.reshape(1, D)
    be1r = be1.reshape(1, D)
    bgcr = bgc.reshape(1, D)
    b2r = b2.reshape(1, D)
    g2r = g2.reshape(1, D)
    be2r = be2.reshape(1, D)

    h0, st1 = _fc1(x, W1, b1r)
    hr, m = _proj(h0, st1, g1r, be1r, Wr, Wn, bgcr)
    aggp = _make_segsum()(m, sd)
    t, st2 = _gc(hr, aggp, aggp, W2, b2r)
    return _fin(t, st2, g2r, be2r, x)
